# baseline (speedup 1.0000x reference)
import functools

import jax
import jax.numpy as jnp
from jax import lax
from jax.experimental import pallas as pl
from jax.experimental.pallas import tpu as pltpu
from jax.experimental.pallas import tpu_sc as plsc

N = 10000
D = 128
H = 64
T = 28
Q = 3
K = 50

NC = 2
NS = 16
L = 16
NW = NC * NS

NP = 10240
DP = 144
F = 96
TP = 32

EB = 128
NB = 80
BLK = 512

EBAT0 = 160
EBAT1 = 0
KBAT0 = 8
KBAT1 = 0


def _edge_body(src_hbm, dst_hbm, xp_hbm, agg_hbm,
               sidx_v, didx_v, rows_v, agg_sh, semi, semg, sems):
    cid = lax.axis_index("c")
    sid = lax.axis_index("s")
    nbat = jnp.where(cid == 0, EBAT0, EBAT1)
    bat0 = jnp.where(cid == 0, sid * EBAT0, NS * EBAT0 + sid * EBAT1)

    def zrow(i, _):
        for c in range(DP // L):
            rows_v[0, i, pl.ds(c * L, L)] = jnp.zeros((L,), jnp.float32)
        return 0
    lax.fori_loop(0, EB, zrow, 0)
    zrows = NP // NS
    for z in range(zrows // EB):
        pltpu.sync_copy(rows_v.at[0],
                        agg_sh.at[pl.ds(sid * zrows + z * EB, EB)])
    plsc.subcore_barrier()

    e0 = bat0 * EB
    @pl.when(nbat > 0)
    def _():
        pltpu.async_copy(src_hbm.at[pl.ds(e0, EB)], sidx_v.at[0], semi)
        pltpu.async_copy(dst_hbm.at[pl.ds(e0, EB)], didx_v.at[0], semi)

    def body(j, _):
        b = j % 2
        base = e0 + j * EB
        pltpu.make_async_copy(src_hbm.at[pl.ds(base, EB)],
                              sidx_v.at[b], semi).wait()
        pltpu.make_async_copy(dst_hbm.at[pl.ds(base, EB)],
                              didx_v.at[b], semi).wait()

        @pl.when(j + 1 < nbat)
        def _():
            pltpu.async_copy(src_hbm.at[pl.ds(base + EB, EB)],
                             sidx_v.at[1 - b], semi)
            pltpu.async_copy(dst_hbm.at[pl.ds(base + EB, EB)],
                             didx_v.at[1 - b], semi)

        @pl.when(j >= 2)
        def _():
            pltpu.make_async_copy(rows_v.at[b],
                                  agg_sh.at[pl.ds(0, EB)], sems).wait()
        pltpu.async_copy(xp_hbm.at[sidx_v.at[b]], rows_v.at[b], semg).wait()
        pltpu.async_copy(rows_v.at[b], agg_sh.at[didx_v.at[b]], sems,
                         add=True)
        return 0
    lax.fori_loop(0, nbat, body, 0)

    @pl.when(nbat >= 2)
    def _():
        pltpu.make_async_copy(rows_v.at[0], agg_sh.at[pl.ds(0, EB)],
                              sems).wait()
        pltpu.make_async_copy(rows_v.at[1], agg_sh.at[pl.ds(0, EB)],
                              sems).wait()
    plsc.subcore_barrier()

    pltpu.sync_copy(agg_sh.at[pl.ds(sid * zrows, zrows)],
                    agg_hbm.at[cid, pl.ds(sid * zrows, zrows)])


def _dense_body(xp_ref, agg_ref, sc_ref, msk_ref, valid_ref,
                ws_ref, wn_ref, wp_ref, b_ref, r_ref,
                wt_ref, base_ref, w_ref):
    a = agg_ref[0] + agg_ref[1]
    deg = jnp.maximum(a[:, D:D + 1], 1.0)
    agg = a[:, :D] / deg
    xb = xp_ref[...][:, :D]
    h = jnp.maximum(xb @ ws_ref[...] + agg @ wn_ref[...], 0.0)
    out96 = h @ wp_ref[...] + b_ref[...]
    sc = sc_ref[...]
    scb = sc @ r_ref[...]
    inv = (1.0 / sc) @ r_ref[...]
    m = msk_ref[...] > 0.0
    wt_ref[...] = out96 * scb * valid_ref[...]
    base_ref[...] = jnp.where(m, 0.0, out96)
    w_ref[...] = jnp.where(m, inv, 0.0)


def _kb_body(kbt_hbm, wt_hbm, base_hbm, w_hbm, out_hbm,
             kb_v, acc_v, bb_v, ww_v, sem, sem2):
    cid = lax.axis_index("c")
    sid = lax.axis_index("s")
    nbat = jnp.where(cid == 0, KBAT0, KBAT1)
    blk0 = jnp.where(cid == 0, sid * KBAT0, NS * KBAT0 + sid * KBAT1)

    def batch(j, _):
        bidx = blk0 + j
        nb = bidx * NB
        pltpu.sync_copy(kbt_hbm.at[bidx], kb_v)
        cb = pltpu.async_copy(base_hbm.at[pl.ds(nb, NB)], bb_v, sem2)
        cw = pltpu.async_copy(w_hbm.at[pl.ds(nb, NB)], ww_v, sem2)
        pltpu.async_copy(wt_hbm.at[kb_v.at[0]], acc_v, sem).wait()

        W = 8

        def kfire(k, _):
            pltpu.async_copy(wt_hbm.at[kb_v.at[k]], acc_v, sem, add=True)

            @pl.when(k >= W + 1)
            def _():
                pltpu.make_async_copy(wt_hbm.at[kb_v.at[0]], acc_v,
                                      sem).wait()
            return 0
        lax.fori_loop(1, K, kfire, 0)

        def kdrain(k, _):
            pltpu.make_async_copy(wt_hbm.at[kb_v.at[0]], acc_v, sem).wait()
            return 0
        lax.fori_loop(0, W, kdrain, 0)
        cb.wait()
        cw.wait()

        def comb(i, _):
            for c in range(F // L):
                s = pl.ds(c * L, L)
                acc_v[i, s] = bb_v[i, s] + acc_v[i, s] * ww_v[i, s]
            return 0
        lax.fori_loop(0, NB, comb, 0)
        pltpu.sync_copy(acc_v, out_hbm.at[pl.ds(nb, NB)])
        return 0
    lax.fori_loop(0, nbat, batch, 0)


def kernel(x, edge_index, keybom, scaler, key_aggregation_status,
           W_self, W_neigh, W_proj, b_proj):
    f32 = jnp.float32
    i32 = jnp.int32
    E = edge_index.shape[1]
    EP = NS * (EBAT0 + EBAT1) * EB
    assert EP >= E

    xp = jnp.zeros((NP, DP), f32).at[:N, :D].set(x).at[:N, D].set(1.0)
    srcp = jnp.full((EP,), N, i32).at[:E].set(edge_index[0])
    dstp = jnp.full((EP,), N, i32).at[:E].set(edge_index[1])
    kb = jnp.where(keybom < 0, N, keybom)
    kbt3 = (jnp.full((K, NP), N, i32).at[:, :N].set(kb.T)
            .reshape(K, NP // NB, NB).transpose(1, 0, 2))
    scp = jnp.ones((NP, TP), f32).at[:N, :T].set(scaler)
    mskf = jnp.zeros((NP, 1), f32).at[:N].set(
        (key_aggregation_status > 0).astype(f32))
    validf = jnp.zeros((NP, 1), f32).at[:N, :].set(1.0)
    wp96 = jnp.zeros((H, F), f32).at[:, :T * Q].set(W_proj)
    b96 = jnp.zeros((1, F), f32).at[0, :T * Q].set(b_proj)
    rmat = (jnp.arange(F)[None, :] // Q == jnp.arange(TP)[:, None]).astype(f32)

    mesh = plsc.VectorSubcoreMesh(core_axis_name="c", subcore_axis_name="s",
                                  num_cores=NC, num_subcores=NS)

    edge_fn = pl.kernel(
        _edge_body,
        out_type=jax.ShapeDtypeStruct((NC, NP, DP), f32),
        mesh=mesh,
        compiler_params=pltpu.CompilerParams(use_tc_tiling_on_sc=False),
        scratch_types=[
            pltpu.VMEM((2, EB), i32),
            pltpu.VMEM((2, EB), i32),
            pltpu.VMEM((2, EB, DP), f32),
            pltpu.VMEM_SHARED((NP, DP), f32),
            pltpu.SemaphoreType.DMA,
            pltpu.SemaphoreType.DMA,
            pltpu.SemaphoreType.DMA,
        ],
    )
    agg2 = edge_fn(srcp, dstp, xp)

    grid = NP // BLK
    wt, base, w = pl.pallas_call(
        _dense_body,
        grid=(grid,),
        in_specs=[
            pl.BlockSpec((BLK, DP), lambda i: (i, 0)),
            pl.BlockSpec((NC, BLK, DP), lambda i: (0, i, 0)),
            pl.BlockSpec((BLK, TP), lambda i: (i, 0)),
            pl.BlockSpec((BLK, 1), lambda i: (i, 0)),
            pl.BlockSpec((BLK, 1), lambda i: (i, 0)),
            pl.BlockSpec((D, H), lambda i: (0, 0)),
            pl.BlockSpec((D, H), lambda i: (0, 0)),
            pl.BlockSpec((H, F), lambda i: (0, 0)),
            pl.BlockSpec((1, F), lambda i: (0, 0)),
            pl.BlockSpec((TP, F), lambda i: (0, 0)),
        ],
        out_specs=[
            pl.BlockSpec((BLK, F), lambda i: (i, 0)),
            pl.BlockSpec((BLK, F), lambda i: (i, 0)),
            pl.BlockSpec((BLK, F), lambda i: (i, 0)),
        ],
        out_shape=[
            jax.ShapeDtypeStruct((NP, F), f32),
            jax.ShapeDtypeStruct((NP, F), f32),
            jax.ShapeDtypeStruct((NP, F), f32),
        ],
    )(xp, agg2, scp, mskf, validf, W_self, W_neigh, wp96, b96, rmat)

    kb_fn = pl.kernel(
        _kb_body,
        out_type=jax.ShapeDtypeStruct((NP, F), f32),
        mesh=mesh,
        compiler_params=pltpu.CompilerParams(use_tc_tiling_on_sc=False),
        scratch_types=[
            pltpu.VMEM((K, NB), i32),
            pltpu.VMEM((NB, F), f32),
            pltpu.VMEM((NB, F), f32),
            pltpu.VMEM((NB, F), f32),
            pltpu.SemaphoreType.DMA,
            pltpu.SemaphoreType.DMA,
        ],
    )
    outp = kb_fn(kbt3, wt, base, w)

    return outp[:N, :T * Q].reshape(N, T, Q)


# speedup vs baseline: 1.8591x; 1.8591x over previous
import functools

import jax
import jax.numpy as jnp
from jax import lax
from jax.experimental import pallas as pl
from jax.experimental.pallas import tpu as pltpu
from jax.experimental.pallas import tpu_sc as plsc

N = 10000
D = 128
H = 64
T = 28
Q = 3
K = 50

NC = 2
NS = 16
L = 16
NW = NC * NS

NP = 10240
DP = 144
F = 96
TP = 32

EB = 128
NB = 80
BLK = 512

EBAT0 = 80
EBAT1 = 80
KBAT0 = 4
KBAT1 = 4


def _edge_body(src_hbm, dst_hbm, xp_hbm, agg_hbm,
               sidx_v, didx_v, rows_v, agg_sh, semi, semg, sems):
    cid = lax.axis_index("c")
    sid = lax.axis_index("s")
    nbat = jnp.where(cid == 0, EBAT0, EBAT1)
    bat0 = jnp.where(cid == 0, sid * EBAT0, NS * EBAT0 + sid * EBAT1)

    def zrow(i, _):
        for c in range(DP // L):
            rows_v[0, i, pl.ds(c * L, L)] = jnp.zeros((L,), jnp.float32)
        return 0
    lax.fori_loop(0, EB, zrow, 0)
    zrows = NP // NS
    for z in range(zrows // EB):
        pltpu.sync_copy(rows_v.at[0],
                        agg_sh.at[pl.ds(sid * zrows + z * EB, EB)])
    plsc.subcore_barrier()

    e0 = bat0 * EB
    @pl.when(nbat > 0)
    def _():
        pltpu.async_copy(src_hbm.at[pl.ds(e0, EB)], sidx_v.at[0], semi)
        pltpu.async_copy(dst_hbm.at[pl.ds(e0, EB)], didx_v.at[0], semi)

    def body(j, _):
        b = j % 2
        base = e0 + j * EB
        pltpu.make_async_copy(src_hbm.at[pl.ds(base, EB)],
                              sidx_v.at[b], semi).wait()
        pltpu.make_async_copy(dst_hbm.at[pl.ds(base, EB)],
                              didx_v.at[b], semi).wait()

        @pl.when(j + 1 < nbat)
        def _():
            pltpu.async_copy(src_hbm.at[pl.ds(base + EB, EB)],
                             sidx_v.at[1 - b], semi)
            pltpu.async_copy(dst_hbm.at[pl.ds(base + EB, EB)],
                             didx_v.at[1 - b], semi)

        @pl.when(j >= 2)
        def _():
            pltpu.make_async_copy(rows_v.at[b],
                                  agg_sh.at[pl.ds(0, EB)], sems).wait()
        pltpu.async_copy(xp_hbm.at[sidx_v.at[b]], rows_v.at[b], semg).wait()
        pltpu.async_copy(rows_v.at[b], agg_sh.at[didx_v.at[b]], sems,
                         add=True)
        return 0
    lax.fori_loop(0, nbat, body, 0)

    @pl.when(nbat >= 2)
    def _():
        pltpu.make_async_copy(rows_v.at[0], agg_sh.at[pl.ds(0, EB)],
                              sems).wait()
        pltpu.make_async_copy(rows_v.at[1], agg_sh.at[pl.ds(0, EB)],
                              sems).wait()
    plsc.subcore_barrier()

    pltpu.sync_copy(agg_sh.at[pl.ds(sid * zrows, zrows)],
                    agg_hbm.at[cid, pl.ds(sid * zrows, zrows)])


def _dense_body(xp_ref, agg_ref, sc_ref, msk_ref, valid_ref,
                ws_ref, wn_ref, wp_ref, b_ref, r_ref,
                wt_ref, base_ref, w_ref):
    a = agg_ref[0] + agg_ref[1]
    deg = jnp.maximum(a[:, D:D + 1], 1.0)
    agg = a[:, :D] / deg
    xb = xp_ref[...][:, :D]
    h = jnp.maximum(xb @ ws_ref[...] + agg @ wn_ref[...], 0.0)
    out96 = h @ wp_ref[...] + b_ref[...]
    sc = sc_ref[...]
    scb = sc @ r_ref[...]
    inv = (1.0 / sc) @ r_ref[...]
    m = msk_ref[...] > 0.0
    wt_ref[...] = out96 * scb * valid_ref[...]
    base_ref[...] = jnp.where(m, 0.0, out96)
    w_ref[...] = jnp.where(m, inv, 0.0)


def _kb_body(kbt_hbm, wt_hbm, base_hbm, w_hbm, out_hbm,
             kb_v, acc_v, bb_v, ww_v, wt_sh, sem, sem2):
    cid = lax.axis_index("c")
    sid = lax.axis_index("s")
    nbat = jnp.where(cid == 0, KBAT0, KBAT1)
    blk0 = jnp.where(cid == 0, sid * KBAT0, NS * KBAT0 + sid * KBAT1)

    srows = NP // NS
    pltpu.sync_copy(wt_hbm.at[pl.ds(sid * srows, srows)],
                    wt_sh.at[pl.ds(sid * srows, srows)])
    plsc.subcore_barrier()

    def batch(j, _):
        bidx = blk0 + j
        nb = bidx * NB
        pltpu.sync_copy(kbt_hbm.at[bidx], kb_v)
        cb = pltpu.async_copy(base_hbm.at[pl.ds(nb, NB)], bb_v, sem2)
        cw = pltpu.async_copy(w_hbm.at[pl.ds(nb, NB)], ww_v, sem2)
        pltpu.async_copy(wt_sh.at[kb_v.at[0]], acc_v, sem).wait()

        W = 8

        def kfire(k, _):
            pltpu.async_copy(wt_sh.at[kb_v.at[k]], acc_v, sem, add=True)

            @pl.when(k >= W + 1)
            def _():
                pltpu.make_async_copy(wt_sh.at[kb_v.at[0]], acc_v,
                                      sem).wait()
            return 0
        lax.fori_loop(1, K, kfire, 0)

        def kdrain(k, _):
            pltpu.make_async_copy(wt_sh.at[kb_v.at[0]], acc_v, sem).wait()
            return 0
        lax.fori_loop(0, W, kdrain, 0)
        cb.wait()
        cw.wait()

        def comb(i, _):
            for c in range(F // L):
                s = pl.ds(c * L, L)
                acc_v[i, s] = bb_v[i, s] + acc_v[i, s] * ww_v[i, s]
            return 0
        lax.fori_loop(0, NB, comb, 0)
        pltpu.sync_copy(acc_v, out_hbm.at[pl.ds(nb, NB)])
        return 0
    lax.fori_loop(0, nbat, batch, 0)


def kernel(x, edge_index, keybom, scaler, key_aggregation_status,
           W_self, W_neigh, W_proj, b_proj):
    f32 = jnp.float32
    i32 = jnp.int32
    E = edge_index.shape[1]
    EP = NS * (EBAT0 + EBAT1) * EB
    assert EP >= E

    xp = jnp.zeros((NP, DP), f32).at[:N, :D].set(x).at[:N, D].set(1.0)
    srcp = jnp.full((EP,), N, i32).at[:E].set(edge_index[0])
    dstp = jnp.full((EP,), N, i32).at[:E].set(edge_index[1])
    kb = jnp.where(keybom < 0, N, keybom)
    kbt3 = (jnp.full((K, NP), N, i32).at[:, :N].set(kb.T)
            .reshape(K, NP // NB, NB).transpose(1, 0, 2))
    scp = jnp.ones((NP, TP), f32).at[:N, :T].set(scaler)
    mskf = jnp.zeros((NP, 1), f32).at[:N].set(
        (key_aggregation_status > 0).astype(f32))
    validf = jnp.zeros((NP, 1), f32).at[:N, :].set(1.0)
    wp96 = jnp.zeros((H, F), f32).at[:, :T * Q].set(W_proj)
    b96 = jnp.zeros((1, F), f32).at[0, :T * Q].set(b_proj)
    rmat = (jnp.arange(F)[None, :] // Q == jnp.arange(TP)[:, None]).astype(f32)

    mesh = plsc.VectorSubcoreMesh(core_axis_name="c", subcore_axis_name="s",
                                  num_cores=NC, num_subcores=NS)

    edge_fn = pl.kernel(
        _edge_body,
        out_type=jax.ShapeDtypeStruct((NC, NP, DP), f32),
        mesh=mesh,
        compiler_params=pltpu.CompilerParams(use_tc_tiling_on_sc=False),
        scratch_types=[
            pltpu.VMEM((2, EB), i32),
            pltpu.VMEM((2, EB), i32),
            pltpu.VMEM((2, EB, DP), f32),
            pltpu.VMEM_SHARED((NP, DP), f32),
            pltpu.SemaphoreType.DMA,
            pltpu.SemaphoreType.DMA,
            pltpu.SemaphoreType.DMA,
        ],
    )
    agg2 = edge_fn(srcp, dstp, xp)

    grid = NP // BLK
    wt, base, w = pl.pallas_call(
        _dense_body,
        grid=(grid,),
        in_specs=[
            pl.BlockSpec((BLK, DP), lambda i: (i, 0)),
            pl.BlockSpec((NC, BLK, DP), lambda i: (0, i, 0)),
            pl.BlockSpec((BLK, TP), lambda i: (i, 0)),
            pl.BlockSpec((BLK, 1), lambda i: (i, 0)),
            pl.BlockSpec((BLK, 1), lambda i: (i, 0)),
            pl.BlockSpec((D, H), lambda i: (0, 0)),
            pl.BlockSpec((D, H), lambda i: (0, 0)),
            pl.BlockSpec((H, F), lambda i: (0, 0)),
            pl.BlockSpec((1, F), lambda i: (0, 0)),
            pl.BlockSpec((TP, F), lambda i: (0, 0)),
        ],
        out_specs=[
            pl.BlockSpec((BLK, F), lambda i: (i, 0)),
            pl.BlockSpec((BLK, F), lambda i: (i, 0)),
            pl.BlockSpec((BLK, F), lambda i: (i, 0)),
        ],
        out_shape=[
            jax.ShapeDtypeStruct((NP, F), f32),
            jax.ShapeDtypeStruct((NP, F), f32),
            jax.ShapeDtypeStruct((NP, F), f32),
        ],
    )(xp, agg2, scp, mskf, validf, W_self, W_neigh, wp96, b96, rmat)

    kb_fn = pl.kernel(
        _kb_body,
        out_type=jax.ShapeDtypeStruct((NP, F), f32),
        mesh=mesh,
        compiler_params=pltpu.CompilerParams(use_tc_tiling_on_sc=False),
        scratch_types=[
            pltpu.VMEM((K, NB), i32),
            pltpu.VMEM((NB, F), f32),
            pltpu.VMEM((NB, F), f32),
            pltpu.VMEM((NB, F), f32),
            pltpu.VMEM_SHARED((NP, F), f32),
            pltpu.SemaphoreType.DMA,
            pltpu.SemaphoreType.DMA,
        ],
    )
    outp = kb_fn(kbt3, wt, base, w)

    return outp[:N, :T * Q].reshape(N, T, Q)


# speedup vs baseline: 3.2090x; 1.7261x over previous
import functools

import jax
import jax.numpy as jnp
from jax import lax
from jax.experimental import pallas as pl
from jax.experimental.pallas import tpu as pltpu
from jax.experimental.pallas import tpu_sc as plsc

N = 10000
D = 128
H = 64
T = 28
Q = 3
K = 50

NC = 2
NS = 16
L = 16
NW = NC * NS

NP = 10240
HD = 64
DP2 = 80
F = 96
TP = 32

EB = 128
EP_BATS = 2560
NB = 80
BLK = 512

KBAT0 = 4
KBAT1 = 4


def _edge_body(src_hbm, dst_hbm, xp0_hbm, xp1_hbm, agg_hbm,
               sidx_v, didx_v, rows_v, x_sh, agg_sh, semi, semg, sems):
    cid = lax.axis_index("c")
    sid = lax.axis_index("s")
    nbat = (EP_BATS + NS - 1) // NS
    zrows = NP // NS

    @pl.when(cid == 0)
    def _():
        pltpu.sync_copy(xp0_hbm.at[pl.ds(sid * zrows, zrows)],
                        x_sh.at[pl.ds(sid * zrows, zrows)])

    @pl.when(cid == 1)
    def _():
        pltpu.sync_copy(xp1_hbm.at[pl.ds(sid * zrows, zrows)],
                        x_sh.at[pl.ds(sid * zrows, zrows)])

    def zrow(i, _):
        for c in range(DP2 // L):
            rows_v[0, i, pl.ds(c * L, L)] = jnp.zeros((L,), jnp.float32)
        return 0
    lax.fori_loop(0, EB, zrow, 0)
    for z in range(zrows // EB):
        pltpu.sync_copy(rows_v.at[0],
                        agg_sh.at[pl.ds(sid * zrows + z * EB, EB)])
    plsc.subcore_barrier()

    e0 = sid * nbat * EB
    pltpu.async_copy(src_hbm.at[pl.ds(e0, EB)], sidx_v.at[0], semi)
    pltpu.async_copy(dst_hbm.at[pl.ds(e0, EB)], didx_v.at[0], semi)

    def body(j, _):
        b = j % 2
        base = e0 + j * EB
        pltpu.make_async_copy(src_hbm.at[pl.ds(base, EB)],
                              sidx_v.at[b], semi).wait()
        pltpu.make_async_copy(dst_hbm.at[pl.ds(base, EB)],
                              didx_v.at[b], semi).wait()

        @pl.when(j + 1 < nbat)
        def _():
            pltpu.async_copy(src_hbm.at[pl.ds(base + EB, EB)],
                             sidx_v.at[1 - b], semi)
            pltpu.async_copy(dst_hbm.at[pl.ds(base + EB, EB)],
                             didx_v.at[1 - b], semi)

        @pl.when(j >= 2)
        def _():
            pltpu.make_async_copy(rows_v.at[b],
                                  agg_sh.at[pl.ds(0, EB)], sems).wait()
        pltpu.async_copy(x_sh.at[sidx_v.at[b]], rows_v.at[b], semg).wait()
        pltpu.async_copy(rows_v.at[b], agg_sh.at[didx_v.at[b]], sems,
                         add=True)
        return 0
    lax.fori_loop(0, nbat, body, 0)
    pltpu.make_async_copy(rows_v.at[0], agg_sh.at[pl.ds(0, EB)],
                          sems).wait()
    pltpu.make_async_copy(rows_v.at[1], agg_sh.at[pl.ds(0, EB)],
                          sems).wait()
    plsc.subcore_barrier()

    pltpu.sync_copy(agg_sh.at[pl.ds(sid * zrows, zrows)],
                    agg_hbm.at[cid, pl.ds(sid * zrows, zrows)])


def _dense_body(x0_ref, x1_ref, agg_ref, sc_ref, msk_ref, valid_ref,
                ws0_ref, ws1_ref, wn0_ref, wn1_ref, wp_ref, b_ref, r_ref,
                wt_ref, base_ref, w_ref):
    a0 = agg_ref[0]
    a1 = agg_ref[1]
    deg = jnp.maximum(a0[:, HD:HD + 1], 1.0)
    aggm = (a0[:, :HD] @ wn0_ref[...] + a1[:, :HD] @ wn1_ref[...]) / deg
    xs = (x0_ref[...][:, :HD] @ ws0_ref[...]
          + x1_ref[...][:, :HD] @ ws1_ref[...])
    h = jnp.maximum(xs + aggm, 0.0)
    out96 = h @ wp_ref[...] + b_ref[...]
    sc = sc_ref[...]
    scb = sc @ r_ref[...]
    inv = (1.0 / sc) @ r_ref[...]
    m = msk_ref[...] > 0.0
    wt_ref[...] = out96 * scb * valid_ref[...]
    base_ref[...] = jnp.where(m, 0.0, out96)
    w_ref[...] = jnp.where(m, inv, 0.0)


def _kb_body(kbt_hbm, wt_hbm, base_hbm, w_hbm, out_hbm,
             kb_v, acc_v, bb_v, ww_v, wt_sh, sem, sem2):
    cid = lax.axis_index("c")
    sid = lax.axis_index("s")
    nbat = jnp.where(cid == 0, KBAT0, KBAT1)
    blk0 = jnp.where(cid == 0, sid * KBAT0, NS * KBAT0 + sid * KBAT1)

    srows = NP // NS
    pltpu.sync_copy(wt_hbm.at[pl.ds(sid * srows, srows)],
                    wt_sh.at[pl.ds(sid * srows, srows)])
    plsc.subcore_barrier()

    def batch(j, _):
        bidx = blk0 + j
        nb = bidx * NB
        pltpu.sync_copy(kbt_hbm.at[bidx], kb_v)
        cb = pltpu.async_copy(base_hbm.at[pl.ds(nb, NB)], bb_v, sem2)
        cw = pltpu.async_copy(w_hbm.at[pl.ds(nb, NB)], ww_v, sem2)
        pltpu.async_copy(wt_sh.at[kb_v.at[0]], acc_v, sem).wait()

        W = 8

        def kfire(k, _):
            pltpu.async_copy(wt_sh.at[kb_v.at[k]], acc_v, sem, add=True)

            @pl.when(k >= W + 1)
            def _():
                pltpu.make_async_copy(wt_sh.at[kb_v.at[0]], acc_v,
                                      sem).wait()
            return 0
        lax.fori_loop(1, K, kfire, 0)

        def kdrain(k, _):
            pltpu.make_async_copy(wt_sh.at[kb_v.at[0]], acc_v, sem).wait()
            return 0
        lax.fori_loop(0, W, kdrain, 0)
        cb.wait()
        cw.wait()

        def comb(i, _):
            for c in range(F // L):
                s = pl.ds(c * L, L)
                acc_v[i, s] = bb_v[i, s] + acc_v[i, s] * ww_v[i, s]
            return 0
        lax.fori_loop(0, NB, comb, 0)
        pltpu.sync_copy(acc_v, out_hbm.at[pl.ds(nb, NB)])
        return 0
    lax.fori_loop(0, nbat, batch, 0)


def kernel(x, edge_index, keybom, scaler, key_aggregation_status,
           W_self, W_neigh, W_proj, b_proj):
    f32 = jnp.float32
    i32 = jnp.int32
    E = edge_index.shape[1]
    EP = EP_BATS * EB
    assert EP >= E

    xp0 = (jnp.zeros((NP, DP2), f32).at[:N, :HD].set(x[:, :HD])
           .at[:N, HD].set(1.0))
    xp1 = jnp.zeros((NP, DP2), f32).at[:N, :HD].set(x[:, HD:])
    srcp = jnp.full((EP,), N, i32).at[:E].set(edge_index[0])
    dstp = jnp.full((EP,), N, i32).at[:E].set(edge_index[1])
    kb = jnp.where(keybom < 0, N, keybom)
    kbt3 = (jnp.full((K, NP), N, i32).at[:, :N].set(kb.T)
            .reshape(K, NP // NB, NB).transpose(1, 0, 2))
    scp = jnp.ones((NP, TP), f32).at[:N, :T].set(scaler)
    mskf = jnp.zeros((NP, 1), f32).at[:N].set(
        (key_aggregation_status > 0).astype(f32))
    validf = jnp.zeros((NP, 1), f32).at[:N, :].set(1.0)
    wp96 = jnp.zeros((H, F), f32).at[:, :T * Q].set(W_proj)
    b96 = jnp.zeros((1, F), f32).at[0, :T * Q].set(b_proj)
    rmat = (jnp.arange(F)[None, :] // Q == jnp.arange(TP)[:, None]).astype(f32)

    mesh = plsc.VectorSubcoreMesh(core_axis_name="c", subcore_axis_name="s",
                                  num_cores=NC, num_subcores=NS)

    edge_fn = pl.kernel(
        _edge_body,
        out_type=jax.ShapeDtypeStruct((NC, NP, DP2), f32),
        mesh=mesh,
        compiler_params=pltpu.CompilerParams(use_tc_tiling_on_sc=False),
        scratch_types=[
            pltpu.VMEM((2, EB), i32),
            pltpu.VMEM((2, EB), i32),
            pltpu.VMEM((2, EB, DP2), f32),
            pltpu.VMEM_SHARED((NP, DP2), f32),
            pltpu.VMEM_SHARED((NP, DP2), f32),
            pltpu.SemaphoreType.DMA,
            pltpu.SemaphoreType.DMA,
            pltpu.SemaphoreType.DMA,
        ],
    )
    agg2 = edge_fn(srcp, dstp, xp0, xp1)

    grid = NP // BLK
    wt, base, w = pl.pallas_call(
        _dense_body,
        grid=(grid,),
        in_specs=[
            pl.BlockSpec((BLK, DP2), lambda i: (i, 0)),
            pl.BlockSpec((BLK, DP2), lambda i: (i, 0)),
            pl.BlockSpec((NC, BLK, DP2), lambda i: (0, i, 0)),
            pl.BlockSpec((BLK, TP), lambda i: (i, 0)),
            pl.BlockSpec((BLK, 1), lambda i: (i, 0)),
            pl.BlockSpec((BLK, 1), lambda i: (i, 0)),
            pl.BlockSpec((HD, H), lambda i: (0, 0)),
            pl.BlockSpec((HD, H), lambda i: (0, 0)),
            pl.BlockSpec((HD, H), lambda i: (0, 0)),
            pl.BlockSpec((HD, H), lambda i: (0, 0)),
            pl.BlockSpec((H, F), lambda i: (0, 0)),
            pl.BlockSpec((1, F), lambda i: (0, 0)),
            pl.BlockSpec((TP, F), lambda i: (0, 0)),
        ],
        out_specs=[
            pl.BlockSpec((BLK, F), lambda i: (i, 0)),
            pl.BlockSpec((BLK, F), lambda i: (i, 0)),
            pl.BlockSpec((BLK, F), lambda i: (i, 0)),
        ],
        out_shape=[
            jax.ShapeDtypeStruct((NP, F), f32),
            jax.ShapeDtypeStruct((NP, F), f32),
            jax.ShapeDtypeStruct((NP, F), f32),
        ],
    )(xp0, xp1, agg2, scp, mskf, validf,
      W_self[:HD], W_self[HD:], W_neigh[:HD], W_neigh[HD:],
      wp96, b96, rmat)

    kb_fn = pl.kernel(
        _kb_body,
        out_type=jax.ShapeDtypeStruct((NP, F), f32),
        mesh=mesh,
        compiler_params=pltpu.CompilerParams(use_tc_tiling_on_sc=False),
        scratch_types=[
            pltpu.VMEM((K, NB), i32),
            pltpu.VMEM((NB, F), f32),
            pltpu.VMEM((NB, F), f32),
            pltpu.VMEM((NB, F), f32),
            pltpu.VMEM_SHARED((NP, F), f32),
            pltpu.SemaphoreType.DMA,
            pltpu.SemaphoreType.DMA,
        ],
    )
    outp = kb_fn(kbt3, wt, base, w)

    return outp[:N, :T * Q].reshape(N, T, Q)


# speedup vs baseline: 3.8747x; 1.2075x over previous
import jax
import jax.numpy as jnp
from jax import lax
from jax.experimental import pallas as pl
from jax.experimental.pallas import tpu as pltpu
from jax.experimental.pallas import tpu_sc as plsc

N = 10000
D = 128
H = 64
T = 28
Q = 3
K = 50

NC = 2
NS = 16
L = 16
NW = NC * NS

NP = 10240
HD = 64
DP2 = 80
F = 128
FS = 96
TP = 32
K2 = 56

EB = 128
NB = 128
BLK = 512

EBATS = 2500
EBS = EBATS // NS
EBR = EBATS - EBS * NS
KBAT0 = 3
KBAT1 = 2


def _edge_body(src_hbm, dst_hbm, x_hbm, valid_hbm, agg_hbm,
               sidx_v, didx_v, rows_v, x_sh, agg_sh, semi, semg, sems):
    cid = lax.axis_index("c")
    sid = lax.axis_index("s")
    nbat = EBS + jnp.where(sid < EBR, 1, 0)
    bat0 = sid * EBS + jnp.minimum(sid, EBR)
    zrows = NP // NS

    def zrow(i, _):
        for c in range(DP2 // L):
            rows_v[0, i, pl.ds(c * L, L)] = jnp.zeros((L,), jnp.float32)
        return 0
    lax.fori_loop(0, EB, zrow, 0)
    for z in range(zrows // EB):
        pltpu.sync_copy(rows_v.at[0],
                        agg_sh.at[pl.ds(sid * zrows + z * EB, EB)])
        pltpu.sync_copy(rows_v.at[0],
                        x_sh.at[pl.ds(sid * zrows + z * EB, EB)])
    r0 = sid * zrows

    @pl.when(cid == 0)
    def _():
        pltpu.sync_copy(x_hbm.at[pl.ds(r0, zrows), pl.ds(0, HD)],
                        x_sh.at[pl.ds(r0, zrows), pl.ds(0, HD)])

    @pl.when(cid == 1)
    def _():
        pltpu.sync_copy(x_hbm.at[pl.ds(r0, zrows), pl.ds(HD, HD)],
                        x_sh.at[pl.ds(r0, zrows), pl.ds(0, HD)])
    pltpu.sync_copy(valid_hbm.at[pl.ds(r0, zrows)],
                    x_sh.at[pl.ds(r0, zrows), pl.ds(HD, 8)])
    plsc.subcore_barrier()

    e0 = bat0 * EB
    pltpu.async_copy(src_hbm.at[pl.ds(e0, EB)], sidx_v.at[0], semi)
    pltpu.async_copy(dst_hbm.at[pl.ds(e0, EB)], didx_v.at[0], semi)

    def body(j, _):
        b = j % 2
        base = e0 + j * EB
        pltpu.make_async_copy(src_hbm.at[pl.ds(base, EB)],
                              sidx_v.at[b], semi).wait()
        pltpu.make_async_copy(dst_hbm.at[pl.ds(base, EB)],
                              didx_v.at[b], semi).wait()

        @pl.when(j + 1 < nbat)
        def _():
            pltpu.async_copy(src_hbm.at[pl.ds(base + EB, EB)],
                             sidx_v.at[1 - b], semi)
            pltpu.async_copy(dst_hbm.at[pl.ds(base + EB, EB)],
                             didx_v.at[1 - b], semi)

        @pl.when(j >= 2)
        def _():
            pltpu.make_async_copy(rows_v.at[b],
                                  agg_sh.at[pl.ds(0, EB)], sems).wait()
        pltpu.async_copy(x_sh.at[sidx_v.at[b]], rows_v.at[b], semg).wait()
        pltpu.async_copy(rows_v.at[b], agg_sh.at[didx_v.at[b]], sems,
                         add=True)
        return 0
    lax.fori_loop(0, nbat, body, 0)
    pltpu.make_async_copy(rows_v.at[0], agg_sh.at[pl.ds(0, EB)],
                          sems).wait()
    pltpu.make_async_copy(rows_v.at[1], agg_sh.at[pl.ds(0, EB)],
                          sems).wait()
    plsc.subcore_barrier()

    pltpu.sync_copy(agg_sh.at[pl.ds(r0, zrows)],
                    agg_hbm.at[cid, pl.ds(r0, zrows), pl.ds(0, DP2)])


def _dense_body(x_ref, agg_ref, sc_ref, msk_ref, valid_ref,
                ws0_ref, ws1_ref, wn0_ref, wn1_ref, wp_ref, b_ref, r_ref,
                wt_ref, base_ref, w_ref):
    a0 = agg_ref[0]
    a1 = agg_ref[1]
    deg = jnp.maximum(a0[:, HD:HD + 1], 1.0)
    aggm = (a0[:, :HD] @ wn0_ref[...] + a1[:, :HD] @ wn1_ref[...]) / deg
    xb = x_ref[...]
    xs = xb[:, :HD] @ ws0_ref[...] + xb[:, HD:] @ ws1_ref[...]
    h = jnp.maximum(xs + aggm, 0.0)
    out96 = h @ wp_ref[...] + b_ref[...]
    sc = sc_ref[...]
    scb = sc @ r_ref[...]
    inv = (1.0 / sc) @ r_ref[...]
    m = msk_ref[...] > 0.0
    wt_ref[...] = out96 * scb * valid_ref[...]
    base_ref[...] = jnp.where(m, 0.0, out96)
    w_ref[...] = jnp.where(m, inv, 0.0)


def _kb_body(kbt_hbm, wt_hbm, base_hbm, w_hbm, out_hbm,
             kb_v, acc_v, bb_v, ww_v, wt_sh, sem, sem2):
    cid = lax.axis_index("c")
    sid = lax.axis_index("s")
    nbat = jnp.where(cid == 0, KBAT0, KBAT1)
    blk0 = jnp.where(cid == 0, sid * KBAT0, NS * KBAT0 + sid * KBAT1)

    srows = NP // NS
    pltpu.sync_copy(wt_hbm.at[pl.ds(sid * srows, srows), pl.ds(0, FS)],
                    wt_sh.at[pl.ds(sid * srows, srows)])
    plsc.subcore_barrier()

    def batch(j, _):
        bidx = blk0 + j
        nb = bidx * NB
        pltpu.sync_copy(kbt_hbm.at[bidx, pl.ds(0, K)], kb_v)
        cb = pltpu.async_copy(base_hbm.at[pl.ds(nb, NB), pl.ds(0, FS)],
                              bb_v, sem2)
        cw = pltpu.async_copy(w_hbm.at[pl.ds(nb, NB), pl.ds(0, FS)],
                              ww_v, sem2)
        pltpu.async_copy(wt_sh.at[kb_v.at[0]], acc_v, sem).wait()

        W = 8

        def kfire(k, _):
            pltpu.async_copy(wt_sh.at[kb_v.at[k]], acc_v, sem, add=True)

            @pl.when(k >= W + 1)
            def _():
                pltpu.make_async_copy(wt_sh.at[kb_v.at[0]], acc_v,
                                      sem).wait()
            return 0
        lax.fori_loop(1, K, kfire, 0)

        def kdrain(k, _):
            pltpu.make_async_copy(wt_sh.at[kb_v.at[0]], acc_v, sem).wait()
            return 0
        lax.fori_loop(0, W, kdrain, 0)
        cb.wait()
        cw.wait()

        def comb(i, _):
            for c in range(FS // L):
                s = pl.ds(c * L, L)
                acc_v[i, s] = bb_v[i, s] + acc_v[i, s] * ww_v[i, s]
            return 0
        lax.fori_loop(0, NB, comb, 0)
        pltpu.sync_copy(acc_v, out_hbm.at[pl.ds(nb, NB)])
        return 0
    lax.fori_loop(0, nbat, batch, 0)


def kernel(x, edge_index, keybom, scaler, key_aggregation_status,
           W_self, W_neigh, W_proj, b_proj):
    f32 = jnp.float32
    i32 = jnp.int32
    E = edge_index.shape[1]
    assert E == EBATS * EB

    src = edge_index[0]
    dst = edge_index[1]
    xpad = jnp.pad(x, ((0, NP - N), (0, 0)))
    kb = jnp.where(keybom < 0, N, keybom)
    kbt3 = (jnp.full((K2, NP), N, i32).at[:K, :N].set(kb.T)
            .reshape(K2, NP // NB, NB).transpose(1, 0, 2))
    scp = jnp.ones((NP, TP), f32).at[:N, :T].set(scaler)
    mskf = jnp.zeros((NP, 1), f32).at[:N].set(
        (key_aggregation_status > 0).astype(f32))
    validf = jnp.zeros((NP, 1), f32).at[:N, :].set(1.0)
    validf8 = jnp.zeros((NP, 8), f32).at[:N, 0].set(1.0)
    wp96 = jnp.zeros((H, F), f32).at[:, :T * Q].set(W_proj)
    b96 = jnp.zeros((1, F), f32).at[0, :T * Q].set(b_proj)
    rmat = ((jnp.arange(F)[None, :] // Q == jnp.arange(TP)[:, None])
            & (jnp.arange(F)[None, :] < T * Q)).astype(f32)

    mesh = plsc.VectorSubcoreMesh(core_axis_name="c", subcore_axis_name="s",
                                  num_cores=NC, num_subcores=NS)

    edge_fn = pl.kernel(
        _edge_body,
        out_type=jax.ShapeDtypeStruct((NC, NP, F), f32),
        mesh=mesh,
        compiler_params=pltpu.CompilerParams(use_tc_tiling_on_sc=False),
        scratch_types=[
            pltpu.VMEM((2, EB), i32),
            pltpu.VMEM((2, EB), i32),
            pltpu.VMEM((2, EB, DP2), f32),
            pltpu.VMEM_SHARED((NP, DP2), f32),
            pltpu.VMEM_SHARED((NP, DP2), f32),
            pltpu.SemaphoreType.DMA,
            pltpu.SemaphoreType.DMA,
            pltpu.SemaphoreType.DMA,
        ],
    )
    agg2 = edge_fn(src, dst, xpad, validf8)

    grid = NP // BLK
    wt, base, w = pl.pallas_call(
        _dense_body,
        grid=(grid,),
        in_specs=[
            pl.BlockSpec((BLK, D), lambda i: (i, 0)),
            pl.BlockSpec((NC, BLK, F), lambda i: (0, i, 0)),
            pl.BlockSpec((BLK, TP), lambda i: (i, 0)),
            pl.BlockSpec((BLK, 1), lambda i: (i, 0)),
            pl.BlockSpec((BLK, 1), lambda i: (i, 0)),
            pl.BlockSpec((HD, H), lambda i: (0, 0)),
            pl.BlockSpec((HD, H), lambda i: (0, 0)),
            pl.BlockSpec((HD, H), lambda i: (0, 0)),
            pl.BlockSpec((HD, H), lambda i: (0, 0)),
            pl.BlockSpec((H, F), lambda i: (0, 0)),
            pl.BlockSpec((1, F), lambda i: (0, 0)),
            pl.BlockSpec((TP, F), lambda i: (0, 0)),
        ],
        out_specs=[
            pl.BlockSpec((BLK, F), lambda i: (i, 0)),
            pl.BlockSpec((BLK, F), lambda i: (i, 0)),
            pl.BlockSpec((BLK, F), lambda i: (i, 0)),
        ],
        out_shape=[
            jax.ShapeDtypeStruct((NP, F), f32),
            jax.ShapeDtypeStruct((NP, F), f32),
            jax.ShapeDtypeStruct((NP, F), f32),
        ],
    )(xpad, agg2, scp, mskf, validf,
      W_self[:HD], W_self[HD:], W_neigh[:HD], W_neigh[HD:],
      wp96, b96, rmat)

    kb_fn = pl.kernel(
        _kb_body,
        out_type=jax.ShapeDtypeStruct((NP, FS), f32),
        mesh=mesh,
        compiler_params=pltpu.CompilerParams(use_tc_tiling_on_sc=False),
        scratch_types=[
            pltpu.VMEM((K, NB), i32),
            pltpu.VMEM((NB, FS), f32),
            pltpu.VMEM((NB, FS), f32),
            pltpu.VMEM((NB, FS), f32),
            pltpu.VMEM_SHARED((NP, FS), f32),
            pltpu.SemaphoreType.DMA,
            pltpu.SemaphoreType.DMA,
        ],
    )
    outp = kb_fn(kbt3, wt, base, w)

    return outp[:N, :T * Q].reshape(N, T, Q)


# speedup vs baseline: 3.8995x; 1.0064x over previous
import jax
import jax.numpy as jnp
from jax import lax
from jax.experimental import pallas as pl
from jax.experimental.pallas import tpu as pltpu
from jax.experimental.pallas import tpu_sc as plsc

N = 10000
D = 128
H = 64
T = 28
Q = 3
K = 50

NC = 2
NS = 16
L = 16
NW = NC * NS

NP = 10240
HD = 64
DP2 = 80
F = 128
FS = 96
TP = 32
K2 = 56

EB = 128
NB = 128
BLK = 512

EBATS = 2500
EBS = EBATS // NS
EBR = EBATS - EBS * NS


def _edge_body(src_hbm, dst_hbm, x_hbm, valid_hbm, agg_hbm,
               sidx_v, didx_v, rows_v, x_sh, agg_sh, semi, semg, sems):
    cid = lax.axis_index("c")
    sid = lax.axis_index("s")
    nbat = EBS + jnp.where(sid < EBR, 1, 0)
    bat0 = sid * EBS + jnp.minimum(sid, EBR)
    zrows = NP // NS

    def zrow(i, _):
        for c in range(DP2 // L):
            rows_v[0, i, pl.ds(c * L, L)] = jnp.zeros((L,), jnp.float32)
        return 0
    lax.fori_loop(0, EB, zrow, 0)
    for z in range(zrows // EB):
        pltpu.sync_copy(rows_v.at[0],
                        agg_sh.at[pl.ds(sid * zrows + z * EB, EB)])
        pltpu.sync_copy(rows_v.at[0],
                        x_sh.at[pl.ds(sid * zrows + z * EB, EB)])
    r0 = sid * zrows

    @pl.when(cid == 0)
    def _():
        pltpu.sync_copy(x_hbm.at[pl.ds(r0, zrows), pl.ds(0, HD)],
                        x_sh.at[pl.ds(r0, zrows), pl.ds(0, HD)])

    @pl.when(cid == 1)
    def _():
        pltpu.sync_copy(x_hbm.at[pl.ds(r0, zrows), pl.ds(HD, HD)],
                        x_sh.at[pl.ds(r0, zrows), pl.ds(0, HD)])
    pltpu.sync_copy(valid_hbm.at[pl.ds(r0, zrows)],
                    x_sh.at[pl.ds(r0, zrows), pl.ds(HD, 8)])
    plsc.subcore_barrier()

    e0 = bat0 * EB
    pltpu.async_copy(src_hbm.at[pl.ds(e0, EB)], sidx_v.at[0], semi)
    pltpu.async_copy(dst_hbm.at[pl.ds(e0, EB)], didx_v.at[0], semi)

    def body(j, _):
        b = j % 2
        base = e0 + j * EB
        pltpu.make_async_copy(src_hbm.at[pl.ds(base, EB)],
                              sidx_v.at[b], semi).wait()
        pltpu.make_async_copy(dst_hbm.at[pl.ds(base, EB)],
                              didx_v.at[b], semi).wait()

        @pl.when(j + 1 < nbat)
        def _():
            pltpu.async_copy(src_hbm.at[pl.ds(base + EB, EB)],
                             sidx_v.at[1 - b], semi)
            pltpu.async_copy(dst_hbm.at[pl.ds(base + EB, EB)],
                             didx_v.at[1 - b], semi)

        @pl.when(j >= 2)
        def _():
            pltpu.make_async_copy(rows_v.at[b],
                                  agg_sh.at[pl.ds(0, EB)], sems).wait()
        pltpu.async_copy(x_sh.at[sidx_v.at[b]], rows_v.at[b], semg).wait()
        pltpu.async_copy(rows_v.at[b], agg_sh.at[didx_v.at[b]], sems,
                         add=True)
        return 0
    lax.fori_loop(0, nbat, body, 0)
    pltpu.make_async_copy(rows_v.at[0], agg_sh.at[pl.ds(0, EB)],
                          sems).wait()
    pltpu.make_async_copy(rows_v.at[1], agg_sh.at[pl.ds(0, EB)],
                          sems).wait()
    plsc.subcore_barrier()

    pltpu.sync_copy(agg_sh.at[pl.ds(r0, zrows)],
                    agg_hbm.at[cid, pl.ds(r0, zrows), pl.ds(0, DP2)])


def _dense_body(x_ref, agg_ref, sc_ref, msk_ref, valid_ref,
                ws0_ref, ws1_ref, wn0_ref, wn1_ref, wp_ref, b_ref, r_ref,
                wt_ref, base_ref, w_ref):
    a0 = agg_ref[0]
    a1 = agg_ref[1]
    deg = jnp.maximum(a0[:, HD:HD + 1], 1.0)
    aggm = (a0[:, :HD] @ wn0_ref[...] + a1[:, :HD] @ wn1_ref[...]) / deg
    xb = x_ref[...]
    xs = xb[:, :HD] @ ws0_ref[...] + xb[:, HD:] @ ws1_ref[...]
    h = jnp.maximum(xs + aggm, 0.0)
    out96 = h @ wp_ref[...] + b_ref[...]
    sc = sc_ref[...]
    scb = sc @ r_ref[...]
    inv = (1.0 / sc) @ r_ref[...]
    m = msk_ref[...] > 0.0
    wt_ref[...] = out96 * scb * valid_ref[...]
    base_ref[...] = jnp.where(m, 0.0, out96)
    w_ref[...] = jnp.where(m, inv, 0.0)


def _kb_body(kbt_hbm, wt_hbm, base_hbm, w_hbm, out_hbm,
             kb_v, acc_v, bb_v, ww_v, wt_sh, sem, sem2):
    cid = lax.axis_index("c")
    sid = lax.axis_index("s")
    wid = sid * NC + cid
    nbat = jnp.where(wid < NW // 2, 3, 2)
    blk0 = wid * 2 + jnp.minimum(wid, NW // 2)

    srows = NP // NS
    pltpu.sync_copy(wt_hbm.at[pl.ds(sid * srows, srows), pl.ds(0, FS)],
                    wt_sh.at[pl.ds(sid * srows, srows)])
    plsc.subcore_barrier()

    def batch(j, _):
        bidx = blk0 + j
        nb = bidx * NB
        pltpu.sync_copy(kbt_hbm.at[bidx, pl.ds(0, K)], kb_v)
        cb = pltpu.async_copy(base_hbm.at[pl.ds(nb, NB), pl.ds(0, FS)],
                              bb_v, sem2)
        cw = pltpu.async_copy(w_hbm.at[pl.ds(nb, NB), pl.ds(0, FS)],
                              ww_v, sem2)
        pltpu.async_copy(wt_sh.at[kb_v.at[0]], acc_v, sem).wait()

        W = 8

        def kfire(k, _):
            pltpu.async_copy(wt_sh.at[kb_v.at[k]], acc_v, sem, add=True)

            @pl.when(k >= W + 1)
            def _():
                pltpu.make_async_copy(wt_sh.at[kb_v.at[0]], acc_v,
                                      sem).wait()
            return 0
        lax.fori_loop(1, K, kfire, 0)

        def kdrain(k, _):
            pltpu.make_async_copy(wt_sh.at[kb_v.at[0]], acc_v, sem).wait()
            return 0
        lax.fori_loop(0, W, kdrain, 0)
        cb.wait()
        cw.wait()

        def comb(i, _):
            for c in range(FS // L):
                s = pl.ds(c * L, L)
                acc_v[i, s] = bb_v[i, s] + acc_v[i, s] * ww_v[i, s]
            return 0
        lax.fori_loop(0, NB, comb, 0)
        pltpu.sync_copy(acc_v, out_hbm.at[pl.ds(nb, NB)])
        return 0
    lax.fori_loop(0, nbat, batch, 0)


def kernel(x, edge_index, keybom, scaler, key_aggregation_status,
           W_self, W_neigh, W_proj, b_proj):
    f32 = jnp.float32
    i32 = jnp.int32
    E = edge_index.shape[1]
    assert E == EBATS * EB

    src = edge_index[0]
    dst = edge_index[1]
    xpad = jnp.pad(x, ((0, NP - N), (0, 0)))
    kb = jnp.where(keybom < 0, N, keybom)
    kbt3 = (jnp.full((K2, NP), N, i32).at[:K, :N].set(kb.T)
            .reshape(K2, NP // NB, NB).transpose(1, 0, 2))
    scp = jnp.ones((NP, TP), f32).at[:N, :T].set(scaler)
    mskf = jnp.zeros((NP, 1), f32).at[:N].set(
        (key_aggregation_status > 0).astype(f32))
    validf = jnp.zeros((NP, 1), f32).at[:N, :].set(1.0)
    validf8 = jnp.zeros((NP, 8), f32).at[:N, 0].set(1.0)
    wp96 = jnp.zeros((H, F), f32).at[:, :T * Q].set(W_proj)
    b96 = jnp.zeros((1, F), f32).at[0, :T * Q].set(b_proj)
    rmat = ((jnp.arange(F)[None, :] // Q == jnp.arange(TP)[:, None])
            & (jnp.arange(F)[None, :] < T * Q)).astype(f32)

    mesh = plsc.VectorSubcoreMesh(core_axis_name="c", subcore_axis_name="s",
                                  num_cores=NC, num_subcores=NS)

    edge_fn = pl.kernel(
        _edge_body,
        out_type=jax.ShapeDtypeStruct((NC, NP, F), f32),
        mesh=mesh,
        compiler_params=pltpu.CompilerParams(use_tc_tiling_on_sc=False),
        scratch_types=[
            pltpu.VMEM((2, EB), i32),
            pltpu.VMEM((2, EB), i32),
            pltpu.VMEM((2, EB, DP2), f32),
            pltpu.VMEM_SHARED((NP, DP2), f32),
            pltpu.VMEM_SHARED((NP, DP2), f32),
            pltpu.SemaphoreType.DMA,
            pltpu.SemaphoreType.DMA,
            pltpu.SemaphoreType.DMA,
        ],
    )
    agg2 = edge_fn(src, dst, xpad, validf8)

    grid = NP // BLK
    wt, base, w = pl.pallas_call(
        _dense_body,
        grid=(grid,),
        in_specs=[
            pl.BlockSpec((BLK, D), lambda i: (i, 0)),
            pl.BlockSpec((NC, BLK, F), lambda i: (0, i, 0)),
            pl.BlockSpec((BLK, TP), lambda i: (i, 0)),
            pl.BlockSpec((BLK, 1), lambda i: (i, 0)),
            pl.BlockSpec((BLK, 1), lambda i: (i, 0)),
            pl.BlockSpec((HD, H), lambda i: (0, 0)),
            pl.BlockSpec((HD, H), lambda i: (0, 0)),
            pl.BlockSpec((HD, H), lambda i: (0, 0)),
            pl.BlockSpec((HD, H), lambda i: (0, 0)),
            pl.BlockSpec((H, F), lambda i: (0, 0)),
            pl.BlockSpec((1, F), lambda i: (0, 0)),
            pl.BlockSpec((TP, F), lambda i: (0, 0)),
        ],
        out_specs=[
            pl.BlockSpec((BLK, F), lambda i: (i, 0)),
            pl.BlockSpec((BLK, F), lambda i: (i, 0)),
            pl.BlockSpec((BLK, F), lambda i: (i, 0)),
        ],
        out_shape=[
            jax.ShapeDtypeStruct((NP, F), f32),
            jax.ShapeDtypeStruct((NP, F), f32),
            jax.ShapeDtypeStruct((NP, F), f32),
        ],
    )(xpad, agg2, scp, mskf, validf,
      W_self[:HD], W_self[HD:], W_neigh[:HD], W_neigh[HD:],
      wp96, b96, rmat)

    kb_fn = pl.kernel(
        _kb_body,
        out_type=jax.ShapeDtypeStruct((NP, FS), f32),
        mesh=mesh,
        compiler_params=pltpu.CompilerParams(use_tc_tiling_on_sc=False),
        scratch_types=[
            pltpu.VMEM((K, NB), i32),
            pltpu.VMEM((NB, FS), f32),
            pltpu.VMEM((NB, FS), f32),
            pltpu.VMEM((NB, FS), f32),
            pltpu.VMEM_SHARED((NP, FS), f32),
            pltpu.SemaphoreType.DMA,
            pltpu.SemaphoreType.DMA,
        ],
    )
    outp = kb_fn(kbt3, wt, base, w)

    return outp[:N, :T * Q].reshape(N, T, Q)
